# baseline (speedup 1.0000x reference)
import functools

import jax
import jax.numpy as jnp
from jax import lax
from jax.experimental import pallas as pl
from jax.experimental.pallas import tpu as pltpu
from jax.experimental.pallas import tpu_sc as plsc

B, T, C = 1, 2048, 768
E = 8
DFF = 1536

BK = 128
NBLK = T // BK
NSTEPS = (2 * T) // BK + E - 1
S = NSTEPS * BK

NC, NS = 2, 16
NW = NC * NS
TW = T // NW


def _ln(x, w, b):
    m = jnp.mean(x, axis=-1, keepdims=True)
    v = jnp.mean(jnp.square(x - m), axis=-1, keepdims=True)
    return (x - m) / jnp.sqrt(v + 1e-5) * w + b


def _route_kernel(x_ref, ln1_w_ref, ln1_b_ref, ln2_w_ref, ln2_b_ref,
                  W_att_ref, W_conf_t_ref, W_diff_ref, W_aff_ref,
                  xa_ref, h0_ref, h1_ref, pos0_ref, pos1_ref, eos_ref):
    x = x_ref[...]
    x_ln = _ln(x, ln1_w_ref[...], ln1_b_ref[...])
    att = jnp.dot(x_ln, W_att_ref[...], preferred_element_type=jnp.float32)
    xa = x + att
    xa_ref[...] = xa
    h = _ln(xa, ln2_w_ref[...], ln2_b_ref[...])

    conf = jax.nn.sigmoid(
        jnp.dot(h, W_conf_t_ref[...], preferred_element_type=jnp.float32))
    diff = jax.nn.sigmoid(
        jnp.dot(h, W_diff_ref[...], preferred_element_type=jnp.float32))
    aff = jnp.dot(h, W_aff_ref[...], preferred_element_type=jnp.float32)
    amax = jnp.max(aff, axis=-1, keepdims=True)
    ex = jnp.exp(aff - amax)
    subsidy = ex / jnp.sum(ex, axis=-1, keepdims=True)
    bids = conf * diff + 0.1 * subsidy

    iota = lax.broadcasted_iota(jnp.int32, (T, E), 1)
    m1 = jnp.max(bids, axis=-1, keepdims=True)
    i1 = jnp.min(jnp.where(bids >= m1, iota, E), axis=-1, keepdims=True)
    oh1 = (iota == i1)
    bids2 = jnp.where(oh1, -jnp.inf, bids)
    m2 = jnp.max(bids2, axis=-1, keepdims=True)
    i2 = jnp.min(jnp.where(bids2 >= m2, iota, E), axis=-1, keepdims=True)
    oh2 = (iota == i2)
    u = jnp.exp(m2 - m1)
    w0 = 1.0 / (1.0 + u)
    w1 = u / (1.0 + u)
    h0_ref[...] = h * jnp.sqrt(w0)
    h1_ref[...] = h * jnp.sqrt(w1)

    G = jnp.where(oh1 | oh2, 1.0, 0.0)
    ir = lax.broadcasted_iota(jnp.int32, (BK, BK), 0)
    ic = lax.broadcasted_iota(jnp.int32, (BK, BK), 1)
    tril = jnp.where(ir > ic, 1.0, 0.0)
    off = jnp.zeros((1, E), jnp.float32)
    blocks = []
    for b in range(NBLK):
        Gb = lax.slice(G, (b * BK, 0), ((b + 1) * BK, E))
        pb = jnp.dot(tril, Gb, preferred_element_type=jnp.float32)
        blocks.append(pb + off)
        off = off + jnp.sum(Gb, axis=0, keepdims=True)
    P_excl = jnp.concatenate(blocks, axis=0)
    n = off

    stepsf = jnp.floor((n + (BK - 1)) * (1.0 / BK))
    ir8 = lax.broadcasted_iota(jnp.int32, (E, E), 0)
    ic8 = lax.broadcasted_iota(jnp.int32, (E, E), 1)
    u_incl = jnp.where(ir8 <= ic8, 1.0, 0.0)
    u_excl = jnp.where(ir8 < ic8, 1.0, 0.0)
    stepsb = jnp.broadcast_to(stepsf, (E, E))
    cs_incl = jnp.dot(stepsb, u_incl,
                      preferred_element_type=jnp.float32)[0:1]
    cs_excl = jnp.dot(stepsb, u_excl,
                      preferred_element_type=jnp.float32)[0:1]
    base_row = cs_excl * float(BK)

    tgt = base_row + P_excl
    pos0 = jnp.sum(jnp.where(oh1, tgt, 0.0), axis=-1, keepdims=True)
    pos1 = jnp.sum(jnp.where(oh2, tgt, 0.0), axis=-1, keepdims=True)
    pos0_ref[...] = pos0.astype(jnp.int32)
    pos1_ref[...] = pos1.astype(jnp.int32)

    meta = jnp.concatenate([cs_excl, stepsf], axis=1)
    eos_ref[...] = meta.astype(jnp.int32)


def _stage_a(x2, ln1_w, ln1_b, ln2_w, ln2_b, W_att, W_conf, W_diff, W_aff):
    return pl.pallas_call(
        _route_kernel,
        out_shape=[
            jax.ShapeDtypeStruct((T, C), jnp.float32),
            jax.ShapeDtypeStruct((T, C), jnp.float32),
            jax.ShapeDtypeStruct((T, C), jnp.float32),
            jax.ShapeDtypeStruct((T, 1), jnp.int32),
            jax.ShapeDtypeStruct((T, 1), jnp.int32),
            jax.ShapeDtypeStruct((1, 2 * E), jnp.int32),
        ],
    )(x2, ln1_w.reshape(1, C), ln1_b.reshape(1, C),
      ln2_w.reshape(1, C), ln2_b.reshape(1, C),
      W_att, W_conf.T, W_diff, W_aff)


def _gmm_kernel(meta_ref, hs_ref, W1_ref, W2_ref, out_ref):
    e = pl.program_id(0)
    n = pl.program_id(1)
    start = meta_ref[e]
    nblk = meta_ref[E + e]
    w1 = W1_ref[0].astype(jnp.bfloat16)
    w2 = W2_ref[0].astype(jnp.bfloat16)

    def body(j, _):
        r0 = (start + j) * BK
        hb = hs_ref[pl.ds(r0, BK), :].astype(jnp.bfloat16)
        hid = jnp.dot(hb, w1, preferred_element_type=jnp.float32)
        hid = jnp.square(jnp.maximum(hid, 0.0)).astype(jnp.bfloat16)
        contrib = jnp.dot(hid, w2, preferred_element_type=jnp.float32)

        @pl.when(n == 0)
        def _():
            out_ref[pl.ds(r0, BK), :] = contrib

        @pl.when(n == 1)
        def _():
            out_ref[pl.ds(r0, BK), :] = out_ref[pl.ds(r0, BK), :] + contrib

        return 0

    lax.fori_loop(0, nblk, body, 0)


def _stage_c(meta, h_sorted, W1, W2):
    grid_spec = pltpu.PrefetchScalarGridSpec(
        num_scalar_prefetch=1,
        grid=(E, 2),
        in_specs=[
            pl.BlockSpec((S, C), lambda e, n, meta: (0, 0)),
            pl.BlockSpec((1, C, DFF // 2), lambda e, n, meta: (e, 0, n)),
            pl.BlockSpec((1, DFF // 2, C), lambda e, n, meta: (e, n, 0)),
        ],
        out_specs=pl.BlockSpec((S, C), lambda e, n, meta: (0, 0)),
    )
    return pl.pallas_call(
        _gmm_kernel,
        grid_spec=grid_spec,
        out_shape=jax.ShapeDtypeStruct((S, C), jnp.float32),
    )(meta, h_sorted, W1, W2)


@functools.cache
def _sc_kernels():
    mesh = plsc.VectorSubcoreMesh(core_axis_name="c", subcore_axis_name="s")

    @functools.partial(
        pl.kernel,
        out_type=jax.ShapeDtypeStruct((S, C), jnp.float32),
        mesh=mesh,
        scratch_types=[
            pltpu.VMEM((TW,), jnp.int32),
            pltpu.VMEM((TW,), jnp.int32),
            pltpu.VMEM((TW, C), jnp.float32),
            pltpu.VMEM((TW, C), jnp.float32),
            pltpu.SemaphoreType.DMA,
            pltpu.SemaphoreType.DMA,
            pltpu.SemaphoreType.DMA,
            pltpu.SemaphoreType.DMA,
        ],
    )
    def _sc_dispatch(h0_hbm, h1_hbm, pos0_hbm, pos1_hbm, hs_hbm,
                     idx0_v, idx1_v, rows0_v, rows1_v, s0, s1, s2, s3):
        wid = lax.axis_index("s") * NC + lax.axis_index("c")
        base = wid * TW
        pltpu.sync_copy(pos0_hbm.at[pl.ds(base, TW)], idx0_v)
        pltpu.sync_copy(pos1_hbm.at[pl.ds(base, TW)], idx1_v)
        c0 = pltpu.async_copy(h0_hbm.at[pl.ds(base, TW)], rows0_v, s0)
        c1 = pltpu.async_copy(h1_hbm.at[pl.ds(base, TW)], rows1_v, s1)
        c0.wait()
        w0 = pltpu.async_copy(rows0_v, hs_hbm.at[idx0_v], s2)
        c1.wait()
        w1 = pltpu.async_copy(rows1_v, hs_hbm.at[idx1_v], s3)
        w0.wait()
        w1.wait()

    @functools.partial(
        pl.kernel,
        out_type=[
            jax.ShapeDtypeStruct((T, C), jnp.float32),
            jax.ShapeDtypeStruct((T, C), jnp.float32),
        ],
        mesh=mesh,
        scratch_types=[
            pltpu.VMEM((TW,), jnp.int32),
            pltpu.VMEM((TW,), jnp.int32),
            pltpu.VMEM((TW, C), jnp.float32),
            pltpu.VMEM((TW, C), jnp.float32),
            pltpu.SemaphoreType.DMA,
            pltpu.SemaphoreType.DMA,
            pltpu.SemaphoreType.DMA,
            pltpu.SemaphoreType.DMA,
        ],
    )
    def _sc_combine(eout_hbm, pos0_hbm, pos1_hbm, g0_hbm, g1_hbm,
                    idx0_v, idx1_v, buf0_v, buf1_v, s0, s1, s2, s3):
        wid = lax.axis_index("s") * NC + lax.axis_index("c")
        base = wid * TW
        pltpu.sync_copy(pos0_hbm.at[pl.ds(base, TW)], idx0_v)
        pltpu.sync_copy(pos1_hbm.at[pl.ds(base, TW)], idx1_v)
        g0 = pltpu.async_copy(eout_hbm.at[idx0_v], buf0_v, s0)
        g1 = pltpu.async_copy(eout_hbm.at[idx1_v], buf1_v, s1)
        g0.wait()
        w0 = pltpu.async_copy(buf0_v, g0_hbm.at[pl.ds(base, TW)], s2)
        g1.wait()
        w1 = pltpu.async_copy(buf1_v, g1_hbm.at[pl.ds(base, TW)], s3)
        w0.wait()
        w1.wait()

    return _sc_dispatch, _sc_combine


def _sum3_kernel(xa_ref, g0_ref, g1_ref, out_ref):
    out_ref[...] = xa_ref[...] + g0_ref[...] + g1_ref[...]


def _stage_e(xa, g0, g1):
    return pl.pallas_call(
        _sum3_kernel,
        out_shape=jax.ShapeDtypeStruct((T, C), jnp.float32),
    )(xa, g0, g1)


@jax.jit
def _run(x, ln1_w, ln1_b, ln2_w, ln2_b, W_att, W_conf, W_diff, W_aff, W1, W2):
    x2 = x.reshape(T, C)
    xa, h0, h1, pos0, pos1, meta = _stage_a(
        x2, ln1_w, ln1_b, ln2_w, ln2_b, W_att, W_conf, W_diff, W_aff)
    pos0f = pos0.reshape(T)
    pos1f = pos1.reshape(T)
    sc_dispatch, sc_combine = _sc_kernels()
    h_sorted = sc_dispatch(h0, h1, pos0f, pos1f)
    eout = _stage_c(meta.reshape(2 * E), h_sorted, W1, W2)
    g0, g1 = sc_combine(eout, pos0f, pos1f)
    out = _stage_e(xa, g0, g1)
    return out.reshape(B, T, C)


def kernel(x, idx, ln1_w, ln1_b, ln2_w, ln2_b, W_att, W_conf, W_diff, W_aff,
           W1, W2):
    del idx
    return _run(x, ln1_w, ln1_b, ln2_w, ln2_b, W_att, W_conf, W_diff, W_aff,
                W1, W2)


# speedup vs baseline: 1.0750x; 1.0750x over previous
import functools

import jax
import jax.numpy as jnp
from jax import lax
from jax.experimental import pallas as pl
from jax.experimental.pallas import tpu as pltpu
from jax.experimental.pallas import tpu_sc as plsc

B, T, C = 1, 2048, 768
E = 8
DFF = 1536

BK = 256
NBLK = T // BK
NSTEPS = (2 * T) // BK + E - 1
S = NSTEPS * BK

NC, NS = 2, 16
NW = NC * NS
TW = T // NW
CW = TW // 2


def _ln(x, w, b):
    m = jnp.mean(x, axis=-1, keepdims=True)
    v = jnp.mean(jnp.square(x - m), axis=-1, keepdims=True)
    return (x - m) / jnp.sqrt(v + 1e-5) * w + b


def _route_kernel(x_ref, ln1_w_ref, ln1_b_ref, ln2_w_ref, ln2_b_ref,
                  W_att_ref, W_conf_t_ref, W_diff_ref, W_aff_ref,
                  xa_ref, h0_ref, h1_ref, pos0_ref, pos1_ref, eos_ref):
    x = x_ref[...]
    x_ln = _ln(x, ln1_w_ref[...], ln1_b_ref[...])
    att = jnp.dot(x_ln, W_att_ref[...], preferred_element_type=jnp.float32)
    xa = x + att
    xa_ref[...] = xa
    h = _ln(xa, ln2_w_ref[...], ln2_b_ref[...])

    conf = jax.nn.sigmoid(
        jnp.dot(h, W_conf_t_ref[...], preferred_element_type=jnp.float32))
    diff = jax.nn.sigmoid(
        jnp.dot(h, W_diff_ref[...], preferred_element_type=jnp.float32))
    aff = jnp.dot(h, W_aff_ref[...], preferred_element_type=jnp.float32)
    amax = jnp.max(aff, axis=-1, keepdims=True)
    ex = jnp.exp(aff - amax)
    subsidy = ex / jnp.sum(ex, axis=-1, keepdims=True)
    bids = conf * diff + 0.1 * subsidy

    iota = lax.broadcasted_iota(jnp.int32, (T, E), 1)
    m1 = jnp.max(bids, axis=-1, keepdims=True)
    i1 = jnp.min(jnp.where(bids >= m1, iota, E), axis=-1, keepdims=True)
    oh1 = (iota == i1)
    bids2 = jnp.where(oh1, -jnp.inf, bids)
    m2 = jnp.max(bids2, axis=-1, keepdims=True)
    i2 = jnp.min(jnp.where(bids2 >= m2, iota, E), axis=-1, keepdims=True)
    oh2 = (iota == i2)
    u = jnp.exp(m2 - m1)
    w0 = 1.0 / (1.0 + u)
    w1 = u / (1.0 + u)
    h0_ref[...] = h * jnp.sqrt(w0)
    h1_ref[...] = h * jnp.sqrt(w1)

    G = jnp.where(oh1 | oh2, 1.0, 0.0)
    ir = lax.broadcasted_iota(jnp.int32, (BK, BK), 0)
    ic = lax.broadcasted_iota(jnp.int32, (BK, BK), 1)
    tril = jnp.where(ir > ic, 1.0, 0.0)
    off = jnp.zeros((1, E), jnp.float32)
    blocks = []
    for b in range(NBLK):
        Gb = lax.slice(G, (b * BK, 0), ((b + 1) * BK, E))
        pb = jnp.dot(tril, Gb, preferred_element_type=jnp.float32)
        blocks.append(pb + off)
        off = off + jnp.sum(Gb, axis=0, keepdims=True)
    P_excl = jnp.concatenate(blocks, axis=0)
    n = off

    stepsf = jnp.floor((n + (BK - 1)) * (1.0 / BK))
    ir8 = lax.broadcasted_iota(jnp.int32, (E, E), 0)
    ic8 = lax.broadcasted_iota(jnp.int32, (E, E), 1)
    u_incl = jnp.where(ir8 <= ic8, 1.0, 0.0)
    u_excl = jnp.where(ir8 < ic8, 1.0, 0.0)
    stepsb = jnp.broadcast_to(stepsf, (E, E))
    cs_incl = jnp.dot(stepsb, u_incl,
                      preferred_element_type=jnp.float32)[0:1]
    cs_excl = jnp.dot(stepsb, u_excl,
                      preferred_element_type=jnp.float32)[0:1]
    base_row = cs_excl * float(BK)

    tgt = base_row + P_excl
    pos0 = jnp.sum(jnp.where(oh1, tgt, 0.0), axis=-1, keepdims=True)
    pos1 = jnp.sum(jnp.where(oh2, tgt, 0.0), axis=-1, keepdims=True)
    pos0_ref[...] = pos0.astype(jnp.int32)
    pos1_ref[...] = pos1.astype(jnp.int32)

    meta = jnp.concatenate([cs_excl, stepsf], axis=1)
    eos_ref[...] = meta.astype(jnp.int32)


def _stage_a(x2, ln1_w, ln1_b, ln2_w, ln2_b, W_att, W_conf, W_diff, W_aff):
    return pl.pallas_call(
        _route_kernel,
        out_shape=[
            jax.ShapeDtypeStruct((T, C), jnp.float32),
            jax.ShapeDtypeStruct((T, C), jnp.float32),
            jax.ShapeDtypeStruct((T, C), jnp.float32),
            jax.ShapeDtypeStruct((T, 1), jnp.int32),
            jax.ShapeDtypeStruct((T, 1), jnp.int32),
            jax.ShapeDtypeStruct((1, 2 * E), jnp.int32),
        ],
    )(x2, ln1_w.reshape(1, C), ln1_b.reshape(1, C),
      ln2_w.reshape(1, C), ln2_b.reshape(1, C),
      W_att, W_conf.T, W_diff, W_aff)


def _gmm_kernel(meta_ref, hs_ref, W1_ref, W2_ref, out_ref):
    e = pl.program_id(0)
    n = pl.program_id(1)
    start = meta_ref[e]
    nblk = meta_ref[E + e]

    def body(j, _):
        r0 = (start + j) * BK
        hid = jnp.dot(hs_ref[pl.ds(r0, BK), :], W1_ref[0],
                      preferred_element_type=jnp.float32)
        hid = jnp.square(jnp.maximum(hid, 0.0))
        contrib = jnp.dot(hid, W2_ref[0], preferred_element_type=jnp.float32)

        @pl.when(n == 0)
        def _():
            out_ref[pl.ds(r0, BK), :] = contrib

        @pl.when(n == 1)
        def _():
            out_ref[pl.ds(r0, BK), :] = out_ref[pl.ds(r0, BK), :] + contrib

        return 0

    lax.fori_loop(0, nblk, body, 0)


def _stage_c(meta, h_sorted, W1, W2):
    grid_spec = pltpu.PrefetchScalarGridSpec(
        num_scalar_prefetch=1,
        grid=(E, 2),
        in_specs=[
            pl.BlockSpec((S, C), lambda e, n, meta: (0, 0)),
            pl.BlockSpec((1, C, DFF // 2), lambda e, n, meta: (e, 0, n)),
            pl.BlockSpec((1, DFF // 2, C), lambda e, n, meta: (e, n, 0)),
        ],
        out_specs=pl.BlockSpec((S, C), lambda e, n, meta: (0, 0)),
    )
    return pl.pallas_call(
        _gmm_kernel,
        grid_spec=grid_spec,
        out_shape=jax.ShapeDtypeStruct((S, C), jnp.float32),
    )(meta, h_sorted, W1, W2)


@functools.cache
def _sc_kernels():
    mesh = plsc.VectorSubcoreMesh(core_axis_name="c", subcore_axis_name="s")

    @functools.partial(
        pl.kernel,
        out_type=jax.ShapeDtypeStruct((S, C), jnp.float32),
        mesh=mesh,
        scratch_types=[
            pltpu.VMEM((TW,), jnp.int32),
            pltpu.VMEM((TW,), jnp.int32),
            pltpu.VMEM((TW, C), jnp.float32),
            pltpu.VMEM((TW, C), jnp.float32),
            pltpu.SemaphoreType.DMA,
            pltpu.SemaphoreType.DMA,
            pltpu.SemaphoreType.DMA,
            pltpu.SemaphoreType.DMA,
        ],
    )
    def _sc_dispatch(h0_hbm, h1_hbm, pos0_hbm, pos1_hbm, hs_hbm,
                     idx0_v, idx1_v, rows0_v, rows1_v, s0, s1, s2, s3):
        wid = lax.axis_index("s") * NC + lax.axis_index("c")
        base = wid * TW
        pltpu.sync_copy(pos0_hbm.at[pl.ds(base, TW)], idx0_v)
        pltpu.sync_copy(pos1_hbm.at[pl.ds(base, TW)], idx1_v)
        c0 = pltpu.async_copy(h0_hbm.at[pl.ds(base, TW)], rows0_v, s0)
        c1 = pltpu.async_copy(h1_hbm.at[pl.ds(base, TW)], rows1_v, s1)
        c0.wait()
        w0 = pltpu.async_copy(rows0_v, hs_hbm.at[idx0_v], s2)
        c1.wait()
        w1 = pltpu.async_copy(rows1_v, hs_hbm.at[idx1_v], s3)
        w0.wait()
        w1.wait()

    @functools.partial(
        pl.kernel,
        out_type=jax.ShapeDtypeStruct((T, C), jnp.float32),
        mesh=mesh,
        scratch_types=[
            pltpu.VMEM((CW,), jnp.int32),
            pltpu.VMEM((CW,), jnp.int32),
            pltpu.VMEM((CW, C), jnp.float32),
            pltpu.VMEM((CW, C), jnp.float32),
            pltpu.VMEM((CW, C), jnp.float32),
            pltpu.SemaphoreType.DMA,
            pltpu.SemaphoreType.DMA,
            pltpu.SemaphoreType.DMA,
        ],
    )
    def _sc_combine(eout_hbm, xa_hbm, pos0_hbm, pos1_hbm, out_hbm,
                    idx0_v, idx1_v, b0_v, b1_v, bx_v, s0, s1, s2):
        wid = lax.axis_index("s") * NC + lax.axis_index("c")
        base = wid * TW
        for half in range(2):
            tb = base + half * CW
            pltpu.sync_copy(pos0_hbm.at[pl.ds(tb, CW)], idx0_v)
            pltpu.sync_copy(pos1_hbm.at[pl.ds(tb, CW)], idx1_v)
            g0 = pltpu.async_copy(eout_hbm.at[idx0_v], b0_v, s0)
            g1 = pltpu.async_copy(eout_hbm.at[idx1_v], b1_v, s1)
            gx = pltpu.async_copy(xa_hbm.at[pl.ds(tb, CW)], bx_v, s2)
            g0.wait()
            g1.wait()
            gx.wait()

            def add_body(t, _):
                for j in range(C // 16):
                    sl = pl.ds(j * 16, 16)
                    bx_v[t, sl] = bx_v[t, sl] + b0_v[t, sl] + b1_v[t, sl]
                return 0

            lax.fori_loop(0, CW, add_body, 0)
            pltpu.sync_copy(bx_v, out_hbm.at[pl.ds(tb, CW)])

    return _sc_dispatch, _sc_combine


@jax.jit
def _run(x, ln1_w, ln1_b, ln2_w, ln2_b, W_att, W_conf, W_diff, W_aff, W1, W2):
    x2 = x.reshape(T, C)
    xa, h0, h1, pos0, pos1, meta = _stage_a(
        x2, ln1_w, ln1_b, ln2_w, ln2_b, W_att, W_conf, W_diff, W_aff)
    pos0f = pos0.reshape(T)
    pos1f = pos1.reshape(T)
    sc_dispatch, sc_combine = _sc_kernels()
    h_sorted = sc_dispatch(h0, h1, pos0f, pos1f)
    eout = _stage_c(meta.reshape(2 * E), h_sorted, W1, W2)
    out = sc_combine(eout, xa, pos0f, pos1f)
    return out.reshape(B, T, C)


def kernel(x, idx, ln1_w, ln1_b, ln2_w, ln2_b, W_att, W_conf, W_diff, W_aff,
           W1, W2):
    del idx
    return _run(x, ln1_w, ln1_b, ln2_w, ln2_b, W_att, W_conf, W_diff, W_aff,
                W1, W2)


# speedup vs baseline: 1.0828x; 1.0073x over previous
import functools

import jax
import jax.numpy as jnp
from jax import lax
from jax.experimental import pallas as pl
from jax.experimental.pallas import tpu as pltpu
from jax.experimental.pallas import tpu_sc as plsc

B, T, C = 1, 2048, 768
E = 8
DFF = 1536

BK = 256
NBLK = T // BK
NSTEPS = (2 * T) // BK + E - 1
S = NSTEPS * BK

NC, NS = 2, 16
NW = NC * NS
TW = T // NW
CW = TW // 2


def _ln(x, w, b):
    m = jnp.mean(x, axis=-1, keepdims=True)
    v = jnp.mean(jnp.square(x - m), axis=-1, keepdims=True)
    return (x - m) / jnp.sqrt(v + 1e-5) * w + b


def _route_kernel(x_ref, ln1_w_ref, ln1_b_ref, ln2_w_ref, ln2_b_ref,
                  W_att_ref, W_conf_t_ref, W_diff_ref, W_aff_ref,
                  xa_ref, h0_ref, h1_ref, pos0_ref, pos1_ref, eos_ref):
    x = x_ref[...]
    x_ln = _ln(x, ln1_w_ref[...], ln1_b_ref[...])
    att = jnp.dot(x_ln, W_att_ref[...], preferred_element_type=jnp.float32)
    xa = x + att
    xa_ref[...] = xa
    h = _ln(xa, ln2_w_ref[...], ln2_b_ref[...])

    conf = jax.nn.sigmoid(
        jnp.dot(h, W_conf_t_ref[...], preferred_element_type=jnp.float32))
    diff = jax.nn.sigmoid(
        jnp.dot(h, W_diff_ref[...], preferred_element_type=jnp.float32))
    aff = jnp.dot(h, W_aff_ref[...], preferred_element_type=jnp.float32)
    amax = jnp.max(aff, axis=-1, keepdims=True)
    ex = jnp.exp(aff - amax)
    subsidy = ex / jnp.sum(ex, axis=-1, keepdims=True)
    bids = conf * diff + 0.1 * subsidy

    iota = lax.broadcasted_iota(jnp.int32, (T, E), 1)
    m1 = jnp.max(bids, axis=-1, keepdims=True)
    i1 = jnp.min(jnp.where(bids >= m1, iota, E), axis=-1, keepdims=True)
    oh1 = (iota == i1)
    bids2 = jnp.where(oh1, -jnp.inf, bids)
    m2 = jnp.max(bids2, axis=-1, keepdims=True)
    i2 = jnp.min(jnp.where(bids2 >= m2, iota, E), axis=-1, keepdims=True)
    oh2 = (iota == i2)
    u = jnp.exp(m2 - m1)
    w0 = 1.0 / (1.0 + u)
    w1 = u / (1.0 + u)
    h0_ref[...] = h * jnp.sqrt(w0)
    h1_ref[...] = h * jnp.sqrt(w1)

    G = jnp.where(oh1 | oh2, 1.0, 0.0)
    ir = lax.broadcasted_iota(jnp.int32, (BK, BK), 0)
    ic = lax.broadcasted_iota(jnp.int32, (BK, BK), 1)
    tril = jnp.where(ir > ic, 1.0, 0.0)
    off = jnp.zeros((1, E), jnp.float32)
    blocks = []
    for b in range(NBLK):
        Gb = lax.slice(G, (b * BK, 0), ((b + 1) * BK, E))
        pb = jnp.dot(tril, Gb, preferred_element_type=jnp.float32)
        blocks.append(pb + off)
        off = off + jnp.sum(Gb, axis=0, keepdims=True)
    P_excl = jnp.concatenate(blocks, axis=0)
    n = off

    stepsf = jnp.floor((n + (BK - 1)) * (1.0 / BK))
    ir8 = lax.broadcasted_iota(jnp.int32, (E, E), 0)
    ic8 = lax.broadcasted_iota(jnp.int32, (E, E), 1)
    u_incl = jnp.where(ir8 <= ic8, 1.0, 0.0)
    u_excl = jnp.where(ir8 < ic8, 1.0, 0.0)
    stepsb = jnp.broadcast_to(stepsf, (E, E))
    cs_incl = jnp.dot(stepsb, u_incl,
                      preferred_element_type=jnp.float32)[0:1]
    cs_excl = jnp.dot(stepsb, u_excl,
                      preferred_element_type=jnp.float32)[0:1]
    base_row = cs_excl * float(BK)

    tgt = base_row + P_excl
    pos0 = jnp.sum(jnp.where(oh1, tgt, 0.0), axis=-1, keepdims=True)
    pos1 = jnp.sum(jnp.where(oh2, tgt, 0.0), axis=-1, keepdims=True)
    pos0_ref[...] = pos0.astype(jnp.int32)
    pos1_ref[...] = pos1.astype(jnp.int32)

    meta = jnp.concatenate([cs_excl, stepsf], axis=1)
    eos_ref[...] = meta.astype(jnp.int32)


def _stage_a(x2, ln1_w, ln1_b, ln2_w, ln2_b, W_att, W_conf, W_diff, W_aff):
    return pl.pallas_call(
        _route_kernel,
        out_shape=[
            jax.ShapeDtypeStruct((T, C), jnp.float32),
            jax.ShapeDtypeStruct((T, C), jnp.float32),
            jax.ShapeDtypeStruct((T, C), jnp.float32),
            jax.ShapeDtypeStruct((T, 1), jnp.int32),
            jax.ShapeDtypeStruct((T, 1), jnp.int32),
            jax.ShapeDtypeStruct((1, 2 * E), jnp.int32),
        ],
    )(x2, ln1_w.reshape(1, C), ln1_b.reshape(1, C),
      ln2_w.reshape(1, C), ln2_b.reshape(1, C),
      W_att, W_conf.T, W_diff, W_aff)


def _gmm_kernel(meta_ref, hs_ref, W1_ref, W2_ref, out_ref):
    e = pl.program_id(0)
    n = pl.program_id(1)
    start = meta_ref[e]
    nblk = meta_ref[E + e]

    def body(j, _):
        r0 = (start + j) * BK
        hid = jnp.dot(hs_ref[pl.ds(r0, BK), :], W1_ref[0],
                      preferred_element_type=jnp.float32)
        hid = jnp.square(jnp.maximum(hid, 0.0))
        contrib = jnp.dot(hid, W2_ref[0], preferred_element_type=jnp.float32)

        @pl.when(n == 0)
        def _():
            out_ref[pl.ds(r0, BK), :] = contrib

        @pl.when(n == 1)
        def _():
            out_ref[pl.ds(r0, BK), :] = out_ref[pl.ds(r0, BK), :] + contrib

        return 0

    lax.fori_loop(0, nblk, body, 0)


def _stage_c(meta, h_sorted, W1, W2):
    grid_spec = pltpu.PrefetchScalarGridSpec(
        num_scalar_prefetch=1,
        grid=(E, 2),
        in_specs=[
            pl.BlockSpec((S, C), lambda e, n, meta: (0, 0)),
            pl.BlockSpec((1, C, DFF // 2), lambda e, n, meta: (e, 0, n)),
            pl.BlockSpec((1, DFF // 2, C), lambda e, n, meta: (e, n, 0)),
        ],
        out_specs=pl.BlockSpec((S, C), lambda e, n, meta: (0, 0)),
    )
    return pl.pallas_call(
        _gmm_kernel,
        grid_spec=grid_spec,
        out_shape=jax.ShapeDtypeStruct((S, C), jnp.float32),
    )(meta, h_sorted, W1, W2)


@functools.cache
def _sc_kernels():
    mesh = plsc.VectorSubcoreMesh(core_axis_name="c", subcore_axis_name="s")

    @functools.partial(
        pl.kernel,
        out_type=jax.ShapeDtypeStruct((S, C), jnp.float32),
        mesh=mesh,
        scratch_types=[
            pltpu.VMEM((TW,), jnp.int32),
            pltpu.VMEM((TW,), jnp.int32),
            pltpu.VMEM((TW, C), jnp.float32),
            pltpu.VMEM((TW, C), jnp.float32),
            pltpu.SemaphoreType.DMA,
            pltpu.SemaphoreType.DMA,
            pltpu.SemaphoreType.DMA,
            pltpu.SemaphoreType.DMA,
        ],
    )
    def _sc_dispatch(h0_hbm, h1_hbm, pos0_hbm, pos1_hbm, hs_hbm,
                     idx0_v, idx1_v, rows0_v, rows1_v, s0, s1, s2, s3):
        wid = lax.axis_index("s") * NC + lax.axis_index("c")
        base = wid * TW
        pltpu.sync_copy(pos0_hbm.at[pl.ds(base, TW)], idx0_v)
        pltpu.sync_copy(pos1_hbm.at[pl.ds(base, TW)], idx1_v)
        c0 = pltpu.async_copy(h0_hbm.at[pl.ds(base, TW)], rows0_v, s0)
        c1 = pltpu.async_copy(h1_hbm.at[pl.ds(base, TW)], rows1_v, s1)
        c0.wait()
        w0 = pltpu.async_copy(rows0_v, hs_hbm.at[idx0_v], s2)
        c1.wait()
        w1 = pltpu.async_copy(rows1_v, hs_hbm.at[idx1_v], s3)
        w0.wait()
        w1.wait()

    @functools.partial(
        pl.kernel,
        out_type=[
            jax.ShapeDtypeStruct((T, C), jnp.float32),
            jax.ShapeDtypeStruct((T, C), jnp.float32),
        ],
        mesh=mesh,
        scratch_types=[
            pltpu.VMEM((TW,), jnp.int32),
            pltpu.VMEM((TW,), jnp.int32),
            pltpu.VMEM((TW, C), jnp.float32),
            pltpu.VMEM((TW, C), jnp.float32),
            pltpu.SemaphoreType.DMA,
            pltpu.SemaphoreType.DMA,
            pltpu.SemaphoreType.DMA,
            pltpu.SemaphoreType.DMA,
        ],
    )
    def _sc_combine(eout_hbm, pos0_hbm, pos1_hbm, g0_hbm, g1_hbm,
                    idx0_v, idx1_v, buf0_v, buf1_v, s0, s1, s2, s3):
        wid = lax.axis_index("s") * NC + lax.axis_index("c")
        base = wid * TW
        pltpu.sync_copy(pos0_hbm.at[pl.ds(base, TW)], idx0_v)
        pltpu.sync_copy(pos1_hbm.at[pl.ds(base, TW)], idx1_v)
        g0 = pltpu.async_copy(eout_hbm.at[idx0_v], buf0_v, s0)
        g1 = pltpu.async_copy(eout_hbm.at[idx1_v], buf1_v, s1)
        g0.wait()
        w0 = pltpu.async_copy(buf0_v, g0_hbm.at[pl.ds(base, TW)], s2)
        g1.wait()
        w1 = pltpu.async_copy(buf1_v, g1_hbm.at[pl.ds(base, TW)], s3)
        w0.wait()
        w1.wait()

    return _sc_dispatch, _sc_combine


def _sum3_kernel(xa_ref, g0_ref, g1_ref, out_ref):
    out_ref[...] = xa_ref[...] + g0_ref[...] + g1_ref[...]


def _stage_e(xa, g0, g1):
    return pl.pallas_call(
        _sum3_kernel,
        out_shape=jax.ShapeDtypeStruct((T, C), jnp.float32),
    )(xa, g0, g1)


@jax.jit
def _run(x, ln1_w, ln1_b, ln2_w, ln2_b, W_att, W_conf, W_diff, W_aff, W1, W2):
    x2 = x.reshape(T, C)
    xa, h0, h1, pos0, pos1, meta = _stage_a(
        x2, ln1_w, ln1_b, ln2_w, ln2_b, W_att, W_conf, W_diff, W_aff)
    pos0f = pos0.reshape(T)
    pos1f = pos1.reshape(T)
    sc_dispatch, sc_combine = _sc_kernels()
    h_sorted = sc_dispatch(h0, h1, pos0f, pos1f)
    eout = _stage_c(meta.reshape(2 * E), h_sorted, W1, W2)
    g0, g1 = sc_combine(eout, pos0f, pos1f)
    out = _stage_e(xa, g0, g1)
    return out.reshape(B, T, C)


def kernel(x, idx, ln1_w, ln1_b, ln2_w, ln2_b, W_att, W_conf, W_diff, W_aff,
           W1, W2):
    del idx
    return _run(x, ln1_w, ln1_b, ln2_w, ln2_b, W_att, W_conf, W_diff, W_aff,
                W1, W2)


# speedup vs baseline: 1.0907x; 1.0073x over previous
import functools

import jax
import jax.numpy as jnp
from jax import lax
from jax.experimental import pallas as pl
from jax.experimental.pallas import tpu as pltpu
from jax.experimental.pallas import tpu_sc as plsc

B, T, C = 1, 2048, 768
E = 8
DFF = 1536

BK = 256
NBLK = T // BK
NSTEPS = (2 * T) // BK + E - 1
S = NSTEPS * BK

NC, NS = 2, 16
NW = NC * NS
TW = T // NW


def _ln(x, w, b):
    m = jnp.mean(x, axis=-1, keepdims=True)
    v = jnp.mean(jnp.square(x - m), axis=-1, keepdims=True)
    return (x - m) / jnp.sqrt(v + 1e-5) * w + b


def _route_kernel(x_ref, ln1_w_ref, ln1_b_ref, ln2_w_ref, ln2_b_ref,
                  W_att_ref, W_conf_t_ref, W_diff_ref, W_aff_ref,
                  xa_ref, h0_ref, h1_ref, pos0_ref, pos1_ref, eos_ref):
    x = x_ref[...]
    x_ln = _ln(x, ln1_w_ref[...], ln1_b_ref[...])
    att = jnp.dot(x_ln, W_att_ref[...], preferred_element_type=jnp.float32)
    xa = x + att
    xa_ref[...] = xa
    h = _ln(xa, ln2_w_ref[...], ln2_b_ref[...])

    conf = jax.nn.sigmoid(
        jnp.dot(h, W_conf_t_ref[...], preferred_element_type=jnp.float32))
    diff = jax.nn.sigmoid(
        jnp.dot(h, W_diff_ref[...], preferred_element_type=jnp.float32))
    aff = jnp.dot(h, W_aff_ref[...], preferred_element_type=jnp.float32)
    amax = jnp.max(aff, axis=-1, keepdims=True)
    ex = jnp.exp(aff - amax)
    subsidy = ex / jnp.sum(ex, axis=-1, keepdims=True)
    bids = conf * diff + 0.1 * subsidy

    iota = lax.broadcasted_iota(jnp.int32, (T, E), 1)
    m1 = jnp.max(bids, axis=-1, keepdims=True)
    i1 = jnp.min(jnp.where(bids >= m1, iota, E), axis=-1, keepdims=True)
    oh1 = (iota == i1)
    bids2 = jnp.where(oh1, -jnp.inf, bids)
    m2 = jnp.max(bids2, axis=-1, keepdims=True)
    i2 = jnp.min(jnp.where(bids2 >= m2, iota, E), axis=-1, keepdims=True)
    oh2 = (iota == i2)
    u = jnp.exp(m2 - m1)
    w0 = 1.0 / (1.0 + u)
    w1 = u / (1.0 + u)
    h0_ref[...] = h * jnp.sqrt(w0)
    h1_ref[...] = h * jnp.sqrt(w1)

    G = jnp.where(oh1 | oh2, 1.0, 0.0)
    ir = lax.broadcasted_iota(jnp.int32, (BK, BK), 0)
    ic = lax.broadcasted_iota(jnp.int32, (BK, BK), 1)
    tril = jnp.where(ir > ic, 1.0, 0.0)
    off = jnp.zeros((1, E), jnp.float32)
    blocks = []
    for b in range(NBLK):
        Gb = lax.slice(G, (b * BK, 0), ((b + 1) * BK, E))
        pb = jnp.dot(tril, Gb, preferred_element_type=jnp.float32)
        blocks.append(pb + off)
        off = off + jnp.sum(Gb, axis=0, keepdims=True)
    P_excl = jnp.concatenate(blocks, axis=0)
    n = off

    stepsf = jnp.floor((n + (BK - 1)) * (1.0 / BK))
    ir8 = lax.broadcasted_iota(jnp.int32, (E, E), 0)
    ic8 = lax.broadcasted_iota(jnp.int32, (E, E), 1)
    u_incl = jnp.where(ir8 <= ic8, 1.0, 0.0)
    u_excl = jnp.where(ir8 < ic8, 1.0, 0.0)
    stepsb = jnp.broadcast_to(stepsf, (E, E))
    cs_incl = jnp.dot(stepsb, u_incl,
                      preferred_element_type=jnp.float32)[0:1]
    cs_excl = jnp.dot(stepsb, u_excl,
                      preferred_element_type=jnp.float32)[0:1]
    base_row = cs_excl * float(BK)

    tgt = base_row + P_excl
    pos0 = jnp.sum(jnp.where(oh1, tgt, 0.0), axis=-1, keepdims=True)
    pos1 = jnp.sum(jnp.where(oh2, tgt, 0.0), axis=-1, keepdims=True)
    pos0_ref[...] = pos0.astype(jnp.int32)
    pos1_ref[...] = pos1.astype(jnp.int32)

    meta = jnp.concatenate([cs_excl, stepsf], axis=1)
    eos_ref[...] = meta.astype(jnp.int32)


def _stage_a(x2, ln1_w, ln1_b, ln2_w, ln2_b, W_att, W_conf, W_diff, W_aff):
    return pl.pallas_call(
        _route_kernel,
        out_shape=[
            jax.ShapeDtypeStruct((T, C), jnp.float32),
            jax.ShapeDtypeStruct((T, C), jnp.float32),
            jax.ShapeDtypeStruct((T, C), jnp.float32),
            jax.ShapeDtypeStruct((T, 1), jnp.int32),
            jax.ShapeDtypeStruct((T, 1), jnp.int32),
            jax.ShapeDtypeStruct((1, 2 * E), jnp.int32),
        ],
    )(x2, ln1_w.reshape(1, C), ln1_b.reshape(1, C),
      ln2_w.reshape(1, C), ln2_b.reshape(1, C),
      W_att, W_conf.T, W_diff, W_aff)


def _gmm_kernel(meta_ref, hs_ref, W1_ref, W2_ref, out_ref):
    e = pl.program_id(0)
    n = pl.program_id(1)
    start = meta_ref[e]
    nblk = meta_ref[E + e]

    def body(j, _):
        r0 = (start + j) * BK
        hid = jnp.dot(hs_ref[pl.ds(r0, BK), :], W1_ref[0],
                      preferred_element_type=jnp.float32)
        hid = jnp.square(jnp.maximum(hid, 0.0))
        contrib = jnp.dot(hid, W2_ref[0], preferred_element_type=jnp.float32)

        @pl.when(n == 0)
        def _():
            out_ref[pl.ds(r0, BK), :] = contrib

        @pl.when(n == 1)
        def _():
            out_ref[pl.ds(r0, BK), :] = out_ref[pl.ds(r0, BK), :] + contrib

        return 0

    lax.fori_loop(0, nblk, body, 0)


def _stage_c(meta, h_sorted, W1, W2):
    grid_spec = pltpu.PrefetchScalarGridSpec(
        num_scalar_prefetch=1,
        grid=(E, 2),
        in_specs=[
            pl.BlockSpec((S, C), lambda e, n, meta: (0, 0)),
            pl.BlockSpec((1, C, DFF // 2), lambda e, n, meta: (e, 0, n)),
            pl.BlockSpec((1, DFF // 2, C), lambda e, n, meta: (e, n, 0)),
        ],
        out_specs=pl.BlockSpec((S, C), lambda e, n, meta: (0, 0)),
    )
    return pl.pallas_call(
        _gmm_kernel,
        grid_spec=grid_spec,
        out_shape=jax.ShapeDtypeStruct((S, C), jnp.float32),
    )(meta, h_sorted, W1, W2)


@functools.cache
def _sc_kernels():
    mesh = plsc.VectorSubcoreMesh(core_axis_name="c", subcore_axis_name="s")

    @functools.partial(
        pl.kernel,
        out_type=jax.ShapeDtypeStruct((S, C), jnp.float32),
        mesh=mesh,
        scratch_types=[
            pltpu.VMEM((TW,), jnp.int32),
            pltpu.VMEM((TW,), jnp.int32),
            pltpu.VMEM((TW, C), jnp.float32),
            pltpu.VMEM((TW, C), jnp.float32),
            pltpu.SemaphoreType.DMA,
            pltpu.SemaphoreType.DMA,
            pltpu.SemaphoreType.DMA,
            pltpu.SemaphoreType.DMA,
        ],
    )
    def _sc_dispatch(h0_hbm, h1_hbm, pos0_hbm, pos1_hbm, hs_hbm,
                     idx0_v, idx1_v, rows0_v, rows1_v, s0, s1, s2, s3):
        wid = lax.axis_index("s") * NC + lax.axis_index("c")
        base = wid * TW
        pltpu.sync_copy(pos0_hbm.at[pl.ds(base, TW)], idx0_v)
        pltpu.sync_copy(pos1_hbm.at[pl.ds(base, TW)], idx1_v)
        c0 = pltpu.async_copy(h0_hbm.at[pl.ds(base, TW)], rows0_v, s0)
        c1 = pltpu.async_copy(h1_hbm.at[pl.ds(base, TW)], rows1_v, s1)
        c0.wait()
        w0 = pltpu.async_copy(rows0_v, hs_hbm.at[idx0_v], s2)
        c1.wait()
        w1 = pltpu.async_copy(rows1_v, hs_hbm.at[idx1_v], s3)
        w0.wait()
        w1.wait()

    @functools.partial(
        pl.kernel,
        out_type=[
            jax.ShapeDtypeStruct((T, C), jnp.float32),
            jax.ShapeDtypeStruct((T, C), jnp.float32),
        ],
        mesh=mesh,
        scratch_types=[
            pltpu.VMEM((TW,), jnp.int32),
            pltpu.VMEM((TW,), jnp.int32),
            pltpu.VMEM((TW, C), jnp.float32),
            pltpu.VMEM((TW, C), jnp.float32),
            pltpu.SemaphoreType.DMA,
            pltpu.SemaphoreType.DMA,
            pltpu.SemaphoreType.DMA,
            pltpu.SemaphoreType.DMA,
        ],
    )
    def _sc_combine(eout_hbm, pos0_hbm, pos1_hbm, g0_hbm, g1_hbm,
                    idx0_v, idx1_v, buf0_v, buf1_v, s0, s1, s2, s3):
        wid = lax.axis_index("s") * NC + lax.axis_index("c")
        base = wid * TW
        pltpu.sync_copy(pos0_hbm.at[pl.ds(base, TW)], idx0_v)
        pltpu.sync_copy(pos1_hbm.at[pl.ds(base, TW)], idx1_v)
        g0 = pltpu.async_copy(eout_hbm.at[idx0_v], buf0_v, s0)
        g1 = pltpu.async_copy(eout_hbm.at[idx1_v], buf1_v, s1)
        g0.wait()
        w0 = pltpu.async_copy(buf0_v, g0_hbm.at[pl.ds(base, TW)], s2)
        g1.wait()
        w1 = pltpu.async_copy(buf1_v, g1_hbm.at[pl.ds(base, TW)], s3)
        w0.wait()
        w1.wait()

    return _sc_dispatch, _sc_combine


def _sum3_kernel(xa_ref, g0_ref, g1_ref, out_ref):
    out_ref[...] = xa_ref[...] + g0_ref[...] + g1_ref[...]


def _stage_e(xa, g0, g1):
    return pl.pallas_call(
        _sum3_kernel,
        out_shape=jax.ShapeDtypeStruct((T, C), jnp.float32),
    )(xa, g0, g1)


@jax.jit
def _run(x, ln1_w, ln1_b, ln2_w, ln2_b, W_att, W_conf, W_diff, W_aff, W1, W2):
    x2 = x.reshape(T, C)
    xa, h0, h1, pos0, pos1, meta = _stage_a(
        x2, ln1_w, ln1_b, ln2_w, ln2_b, W_att, W_conf, W_diff, W_aff)
    pos0f = pos0.reshape(T)
    pos1f = pos1.reshape(T)
    sc_dispatch, sc_combine = _sc_kernels()
    h_sorted = sc_dispatch(h0, h1, pos0f, pos1f)
    eout = _stage_c(meta.reshape(2 * E), h_sorted, W1, W2)
    g0, g1 = sc_combine(eout, pos0f, pos1f)
    out = _stage_e(xa, g0, g1)
    return out.reshape(B, T, C)


def kernel(x, idx, ln1_w, ln1_b, ln2_w, ln2_b, W_att, W_conf, W_diff, W_aff,
           W1, W2):
    del idx
    return _run(x, ln1_w, ln1_b, ln2_w, ln2_b, W_att, W_conf, W_diff, W_aff,
                W1, W2)
